# baseline (speedup 1.0000x reference)
import functools

import jax
import jax.numpy as jnp
from jax import lax
from jax.experimental import pallas as pl
from jax.experimental.pallas import tpu as pltpu
from jax.experimental.pallas import tpu_sc as plsc

N_DIMS = 64
SEQ_LEN = 200
BATCH = 4096

NUM_CORES = 2
NUM_SUBCORES = 16
NUM_WORKERS = NUM_CORES * NUM_SUBCORES
ROWS_PER_WORKER = BATCH // NUM_WORKERS
LANES = 16


def kernel(categories, category_table, position_table):
    mesh = plsc.VectorSubcoreMesh(core_axis_name="c", subcore_axis_name="s")

    @functools.partial(
        pl.kernel,
        mesh=mesh,
        compiler_params=pltpu.CompilerParams(use_tc_tiling_on_sc=False),
        out_type=jax.ShapeDtypeStruct((BATCH, SEQ_LEN, N_DIMS), jnp.float32),
        scratch_types=[
            pltpu.VMEM((SEQ_LEN,), jnp.int32),
            pltpu.VMEM((SEQ_LEN, N_DIMS), jnp.float32),
            pltpu.VMEM((SEQ_LEN, N_DIMS), jnp.float32),
            pltpu.SemaphoreType.DMA,
        ],
    )
    def emb_kernel(cat_hbm, table_hbm, pos_hbm, out_hbm, idx_v, rows_v, pos_v, sem):
        wid = lax.axis_index("s") * NUM_CORES + lax.axis_index("c")
        pltpu.sync_copy(pos_hbm, pos_v)

        def body(i, carry):
            b = wid * ROWS_PER_WORKER + i
            pltpu.sync_copy(cat_hbm.at[b], idx_v)
            pltpu.async_copy(table_hbm.at[idx_v], rows_v, sem).wait()

            def add_row(l, c):
                for j in range(N_DIMS // LANES):
                    sl = (l, pl.ds(j * LANES, LANES))
                    rows_v[sl] = rows_v[sl] + pos_v[sl]
                return c

            lax.fori_loop(0, SEQ_LEN, add_row, 0)
            pltpu.sync_copy(rows_v, out_hbm.at[b])
            return carry

        lax.fori_loop(0, ROWS_PER_WORKER, body, 0)

    return emb_kernel(categories, category_table, position_table)


# speedup vs baseline: 1.1913x; 1.1913x over previous
import functools

import jax
import jax.numpy as jnp
from jax import lax
from jax.experimental import pallas as pl
from jax.experimental.pallas import tpu as pltpu
from jax.experimental.pallas import tpu_sc as plsc

N_DIMS = 64
SEQ_LEN = 200
BATCH = 4096

NUM_CORES = 2
NUM_SUBCORES = 16
NUM_WORKERS = NUM_CORES * NUM_SUBCORES
ROWS_PER_WORKER = BATCH // NUM_WORKERS
LANES = 16
NBUF = 2


def kernel(categories, category_table, position_table):
    mesh = plsc.VectorSubcoreMesh(core_axis_name="c", subcore_axis_name="s")

    @functools.partial(
        pl.kernel,
        mesh=mesh,
        compiler_params=pltpu.CompilerParams(use_tc_tiling_on_sc=False),
        out_type=jax.ShapeDtypeStruct((BATCH, SEQ_LEN, N_DIMS), jnp.float32),
        scratch_types=[
            pltpu.VMEM((ROWS_PER_WORKER, SEQ_LEN), jnp.int32),
            pltpu.VMEM((SEQ_LEN, N_DIMS), jnp.float32),
            [pltpu.VMEM((SEQ_LEN, N_DIMS), jnp.float32) for _ in range(NBUF)],
            [pltpu.VMEM((SEQ_LEN, N_DIMS), jnp.float32) for _ in range(NBUF)],
            [pltpu.SemaphoreType.DMA for _ in range(NBUF)],
            [pltpu.SemaphoreType.DMA for _ in range(NBUF)],
        ],
    )
    def emb_kernel(cat_hbm, table_hbm, pos_hbm, out_hbm,
                   idx_all, pos_v, rows, obuf, gsem, wsem):
        wid = lax.axis_index("s") * NUM_CORES + lax.axis_index("c")
        base = wid * ROWS_PER_WORKER
        pltpu.sync_copy(cat_hbm.at[pl.ds(base, ROWS_PER_WORKER)], idx_all)
        pltpu.sync_copy(pos_hbm, pos_v)

        def gather(i, t):
            pltpu.async_copy(table_hbm.at[idx_all.at[i]], rows[t], gsem[t])

        def gather_wait(i, t):
            pltpu.make_async_copy(
                table_hbm.at[idx_all.at[i]], rows[t], gsem[t]).wait()

        def write(i, t):
            pltpu.async_copy(obuf[t], out_hbm.at[base + i], wsem[t])

        def write_wait(i, t):
            pltpu.make_async_copy(obuf[t], out_hbm.at[base + i], wsem[t]).wait()

        for t in range(NBUF):
            gather(t, t)

        def body(j, carry):
            for t in range(NBUF):
                i = j * NBUF + t
                gather_wait(i, t)

                @pl.when(i >= NBUF)
                def _():
                    write_wait(i - NBUF, t)

                def add_row(l, c):
                    for q in range(N_DIMS // LANES):
                        sl = (l, pl.ds(q * LANES, LANES))
                        obuf[t][sl] = rows[t][sl] + pos_v[sl]
                    return c

                lax.fori_loop(0, SEQ_LEN, add_row, 0)
                write(i, t)

                @pl.when(i + NBUF < ROWS_PER_WORKER)
                def _():
                    gather(i + NBUF, t)
            return carry

        lax.fori_loop(0, ROWS_PER_WORKER // NBUF, body, 0)
        for t in range(NBUF):
            write_wait(ROWS_PER_WORKER - NBUF + t, t)

    return emb_kernel(categories, category_table, position_table)
